# baseline (speedup 1.0000x reference)
import jax
import jax.numpy as jnp
from jax import lax
from jax.experimental import pallas as pl
from jax.experimental.pallas import tpu as pltpu
from jax.experimental.pallas import tpu_sc as plsc

N_ACTIONS = 1000
BATCH = 16384

NUM_CORES = 2
NUM_SUBCORES = 16
LANES = 16
NUM_WORKERS = NUM_CORES * NUM_SUBCORES
B_PER_W = BATCH // NUM_WORKERS
CHUNK = 128
N_CHUNKS = B_PER_W // CHUNK


def _idx_body(batch_hbm, qidx_hbm, pairs_v, qidx_v, sem):
  wid = lax.axis_index("s") * NUM_CORES + lax.axis_index("c")
  base = wid * B_PER_W

  pltpu.sync_copy(batch_hbm.at[pl.ds(2 * B_PER_W * wid, 2 * B_PER_W)], pairs_v)

  copies = []
  for j in range(N_CHUNKS):
    for i in range(CHUNK // LANES):
      a0 = pairs_v[pl.ds(2 * CHUNK * j + LANES * i, LANES)]
      a1 = pairs_v[pl.ds(2 * CHUNK * j + CHUNK + LANES * i, LANES)]
      qidx_v[j, pl.ds(LANES * i, LANES)] = a0 * N_ACTIONS + a1
    copies.append(
        pltpu.async_copy(qidx_v.at[j],
                         qidx_hbm.at[pl.ds(base + CHUNK * j, CHUNK)],
                         sem.at[j]))
  for c in copies:
    c.wait()


def _gather_body(qidx_hbm, table_hbm, out_hbm, idx_v, out_v, sem):
  wid = lax.axis_index("s") * NUM_CORES + lax.axis_index("c")
  base = wid * B_PER_W

  pltpu.sync_copy(qidx_hbm.at[pl.ds(base, B_PER_W)], idx_v)

  copies = [
      pltpu.async_copy(table_hbm.at[idx_v.at[pl.ds(CHUNK * j, CHUNK)]],
                       out_v.at[j], sem.at[j])
      for j in range(N_CHUNKS)
  ]
  for j in range(N_CHUNKS):
    copies[j].wait()
    pltpu.sync_copy(out_v.at[j], out_hbm.at[pl.ds(base + CHUNK * j, CHUNK)])


@jax.jit
def kernel(batch, q_table):
  blocks = BATCH // CHUNK
  batch_lin = (batch.astype(jnp.int32).T
               .reshape(2, blocks, CHUNK)
               .transpose(1, 0, 2)
               .reshape(2 * BATCH))
  mesh = plsc.VectorSubcoreMesh(
      core_axis_name="c", subcore_axis_name="s", num_cores=NUM_CORES)
  idx_pass = pl.kernel(
      _idx_body,
      out_type=jax.ShapeDtypeStruct((BATCH,), jnp.int32),
      mesh=mesh,
      scratch_types=[
          pltpu.VMEM((2 * B_PER_W,), jnp.int32),
          pltpu.VMEM((N_CHUNKS, CHUNK), jnp.int32),
          pltpu.SemaphoreType.DMA((N_CHUNKS,)),
      ],
  )
  gather_pass = pl.kernel(
      _gather_body,
      out_type=jax.ShapeDtypeStruct((BATCH,), jnp.float32),
      mesh=mesh,
      scratch_types=[
          pltpu.VMEM((B_PER_W,), jnp.int32),
          pltpu.VMEM((N_CHUNKS, CHUNK), jnp.float32),
          pltpu.SemaphoreType.DMA((N_CHUNKS,)),
      ],
  )
  qidx = idx_pass(batch_lin)
  table_flat = q_table.reshape(-1)
  return gather_pass(qidx, table_flat)


# speedup vs baseline: 1.0609x; 1.0609x over previous
import jax
import jax.numpy as jnp
from jax import lax
from jax.experimental import pallas as pl
from jax.experimental.pallas import tpu as pltpu
from jax.experimental.pallas import tpu_sc as plsc

N_ACTIONS = 1000
BATCH = 16384

NUM_CORES = 2
NUM_SUBCORES = 16
LANES = 16
NUM_WORKERS = NUM_CORES * NUM_SUBCORES
B_PER_W = BATCH // NUM_WORKERS
CHUNK = 128
N_CHUNKS = B_PER_W // CHUNK


def _sc_body(batch_hbm, table_hbm, out_hbm, pairs_v, qidx_v, out_v, sem):
  wid = lax.axis_index("s") * NUM_CORES + lax.axis_index("c")
  base = wid * B_PER_W

  pltpu.sync_copy(batch_hbm.at[pl.ds(2 * B_PER_W * wid, 2 * B_PER_W)], pairs_v)

  copies = []
  for j in range(N_CHUNKS):
    for i in range(CHUNK // LANES):
      a0 = pairs_v[pl.ds(2 * CHUNK * j + LANES * i, LANES)]
      a1 = pairs_v[pl.ds(2 * CHUNK * j + CHUNK + LANES * i, LANES)]
      qidx_v[j, pl.ds(LANES * i, LANES)] = a0 * N_ACTIONS + a1
    copies.append(
        pltpu.async_copy(table_hbm.at[qidx_v.at[j]], out_v.at[j], sem.at[j]))

  for j in range(N_CHUNKS):
    copies[j].wait()
    pltpu.sync_copy(out_v.at[j], out_hbm.at[pl.ds(base + CHUNK * j, CHUNK)])


@jax.jit
def kernel(batch, q_table):
  blocks = BATCH // CHUNK
  batch_lin = (batch.astype(jnp.int32).T
               .reshape(2, blocks, CHUNK)
               .transpose(1, 0, 2)
               .reshape(2 * BATCH))
  table_flat = q_table.reshape(-1)
  mesh = plsc.VectorSubcoreMesh(
      core_axis_name="c", subcore_axis_name="s", num_cores=NUM_CORES)
  run = pl.kernel(
      _sc_body,
      out_type=jax.ShapeDtypeStruct((BATCH,), jnp.float32),
      mesh=mesh,
      scratch_types=[
          pltpu.VMEM((2 * B_PER_W,), jnp.int32),
          pltpu.VMEM((N_CHUNKS, CHUNK), jnp.int32),
          pltpu.VMEM((N_CHUNKS, CHUNK), jnp.float32),
          pltpu.SemaphoreType.DMA((N_CHUNKS,)),
      ],
  )
  return run(batch_lin, table_flat)
